# baseline (speedup 1.0000x reference)
import functools

import jax
import jax.numpy as jnp
from jax.experimental import pallas as pl
from jax.experimental.pallas import tpu as pltpu

_INV_TEMP = 10.0
_EPS = 1e-12

_BN = 256


def _logits_kernel(x_ref, g_ref, o_ref, xn_ref):
    @pl.when(pl.program_id(0) == 0)
    def _():
        x = x_ref[...]
        xs = x * (_INV_TEMP / jnp.maximum(jnp.sqrt(jnp.sum(x * x, axis=1, keepdims=True)), _EPS))
        xn_ref[...] = xs.astype(jnp.bfloat16)

    g = g_ref[...]
    gn = g / jnp.maximum(jnp.sqrt(jnp.sum(g * g, axis=1, keepdims=True)), _EPS)
    o_ref[...] = jax.lax.dot_general(
        xn_ref[...],
        gn.astype(jnp.bfloat16),
        (((1,), (1,)), ((), ())),
        preferred_element_type=jnp.float32,
    )


@functools.partial(jax.jit, static_argnames=())
def kernel(x, group_features):
    m, k = x.shape
    n, _ = group_features.shape
    grid = (n // _BN,)
    return pl.pallas_call(
        _logits_kernel,
        grid=grid,
        in_specs=[
            pl.BlockSpec((m, k), lambda j: (0, 0)),
            pl.BlockSpec((_BN, k), lambda j: (j, 0)),
        ],
        out_specs=pl.BlockSpec((m, _BN), lambda j: (0, j)),
        out_shape=jax.ShapeDtypeStruct((m, n), jnp.float32),
        scratch_shapes=[pltpu.VMEM((m, k), jnp.bfloat16)],
        compiler_params=pltpu.CompilerParams(
            dimension_semantics=("arbitrary",),
        ),
    )(x, group_features)


# speedup vs baseline: 1.1663x; 1.1663x over previous
import functools

import jax
import jax.numpy as jnp
from jax.experimental import pallas as pl
from jax.experimental.pallas import tpu as pltpu

_INV_TEMP = 10.0
_EPS = 1e-12

_BN = 1024


def _logits_kernel(x_ref, g_ref, o_ref, xn_ref):
    @pl.when(pl.program_id(0) == 0)
    def _():
        x = x_ref[...]
        xs = x * (_INV_TEMP / jnp.maximum(jnp.sqrt(jnp.sum(x * x, axis=1, keepdims=True)), _EPS))
        xn_ref[...] = xs.astype(jnp.bfloat16)

    g = g_ref[...]
    gn = g / jnp.maximum(jnp.sqrt(jnp.sum(g * g, axis=1, keepdims=True)), _EPS)
    o_ref[...] = jax.lax.dot_general(
        xn_ref[...],
        gn.astype(jnp.bfloat16),
        (((1,), (1,)), ((), ())),
        preferred_element_type=jnp.float32,
    )


@functools.partial(jax.jit, static_argnames=())
def kernel(x, group_features):
    m, k = x.shape
    n, _ = group_features.shape
    grid = (n // _BN,)
    return pl.pallas_call(
        _logits_kernel,
        grid=grid,
        in_specs=[
            pl.BlockSpec((m, k), lambda j: (0, 0)),
            pl.BlockSpec((_BN, k), lambda j: (j, 0)),
        ],
        out_specs=pl.BlockSpec((m, _BN), lambda j: (0, j)),
        out_shape=jax.ShapeDtypeStruct((m, n), jnp.float32),
        scratch_shapes=[pltpu.VMEM((m, k), jnp.bfloat16)],
        compiler_params=pltpu.CompilerParams(
            dimension_semantics=("arbitrary",),
        ),
    )(x, group_features)


# speedup vs baseline: 1.1833x; 1.0146x over previous
import functools

import jax
import jax.numpy as jnp
from jax.experimental import pallas as pl
from jax.experimental.pallas import tpu as pltpu

_INV_TEMP = 10.0
_EPS = 1e-12

_BN = 512


def _logits_kernel(x_ref, g_ref, o_ref, xn_ref):
    @pl.when(pl.program_id(0) == 0)
    def _():
        x = x_ref[...]
        xs = x * (_INV_TEMP / jnp.maximum(jnp.sqrt(jnp.sum(x * x, axis=1, keepdims=True)), _EPS))
        xn_ref[...] = xs.astype(jnp.bfloat16)

    g = g_ref[...]
    gn = g / jnp.maximum(jnp.sqrt(jnp.sum(g * g, axis=1, keepdims=True)), _EPS)
    o_ref[...] = jax.lax.dot_general(
        xn_ref[...],
        gn.astype(jnp.bfloat16),
        (((1,), (1,)), ((), ())),
        preferred_element_type=jnp.float32,
    )


@functools.partial(jax.jit, static_argnames=())
def kernel(x, group_features):
    m, k = x.shape
    n, _ = group_features.shape
    grid = (n // _BN,)
    return pl.pallas_call(
        _logits_kernel,
        grid=grid,
        in_specs=[
            pl.BlockSpec((m, k), lambda j: (0, 0)),
            pl.BlockSpec((_BN, k), lambda j: (j, 0)),
        ],
        out_specs=pl.BlockSpec((m, _BN), lambda j: (0, j)),
        out_shape=jax.ShapeDtypeStruct((m, n), jnp.float32),
        scratch_shapes=[pltpu.VMEM((m, k), jnp.bfloat16)],
        compiler_params=pltpu.CompilerParams(
            dimension_semantics=("arbitrary",),
        ),
    )(x, group_features)


# speedup vs baseline: 1.2186x; 1.0298x over previous
import functools

import jax
import jax.numpy as jnp
from jax.experimental import pallas as pl
from jax.experimental.pallas import tpu as pltpu

_INV_TEMP = 10.0
_EPS = 1e-12

_BN = 512


def _logits_kernel(x_ref, g_ref, o_ref):
    x = x_ref[...]
    xs = x * (_INV_TEMP / jnp.maximum(jnp.sqrt(jnp.sum(x * x, axis=1, keepdims=True)), _EPS))
    g = g_ref[...]
    gn = g / jnp.maximum(jnp.sqrt(jnp.sum(g * g, axis=1, keepdims=True)), _EPS)
    o_ref[...] = jax.lax.dot_general(
        xs.astype(jnp.bfloat16),
        gn.astype(jnp.bfloat16),
        (((1,), (1,)), ((), ())),
        preferred_element_type=jnp.float32,
    )


@functools.partial(jax.jit, static_argnames=())
def kernel(x, group_features):
    m, k = x.shape
    n, _ = group_features.shape
    grid = (n // _BN,)
    return pl.pallas_call(
        _logits_kernel,
        grid=grid,
        in_specs=[
            pl.BlockSpec((m, k), lambda j: (0, 0)),
            pl.BlockSpec((_BN, k), lambda j: (j, 0)),
        ],
        out_specs=pl.BlockSpec((m, _BN), lambda j: (0, j)),
        out_shape=jax.ShapeDtypeStruct((m, n), jnp.float32),
        compiler_params=pltpu.CompilerParams(
            dimension_semantics=("parallel",),
        ),
    )(x, group_features)
